# Initial kernel scaffold; baseline (speedup 1.0000x reference)
import jax
import jax.numpy as jnp
from jax.experimental import pallas as pl


def kernel(x, edge_index, adj_values, weight, biases):
    raise NotImplementedError("write your pallas kernel here")


# speedup vs baseline: 1.0840x; 1.0840x over previous
import functools

import jax
import jax.numpy as jnp
from jax.experimental import pallas as pl
from jax.experimental.pallas import tpu as pltpu

N = 49152
E = 393216
D = 128
K = 3


def _matmul_body(xs_ref, w_ref, b_ref, o_ref):
    o_ref[...] = (
        jnp.dot(xs_ref[...], w_ref[...], preferred_element_type=jnp.float32)
        + b_ref[...]
    )


def _final_matmul(xs, weight, biases):
    m = xs.shape[0]
    bm = 1024
    return pl.pallas_call(
        _matmul_body,
        out_shape=jax.ShapeDtypeStruct((m, D), jnp.float32),
        grid=(m // bm,),
        in_specs=[
            pl.BlockSpec((bm, D * K), lambda i: (i, 0)),
            pl.BlockSpec((D * K, D), lambda i: (0, 0)),
            pl.BlockSpec((D,), lambda i: (0,)),
        ],
        out_specs=pl.BlockSpec((bm, D), lambda i: (i, 0)),
    )(xs, weight, biases)


def _spmm(row, col, w, xmat):
    return jnp.zeros(xmat.shape, xmat.dtype).at[row].add(w[:, None] * xmat[col])


def kernel(x, edge_index, adj_values, weight, biases):
    b, n, d = x.shape
    row = edge_index[0]
    col = edge_index[1]
    deg = jnp.zeros((n,), adj_values.dtype).at[row].add(adj_values)
    w = adj_values / jnp.clip(deg[row], 1e-6, None)
    x0 = jnp.transpose(x, (1, 0, 2)).reshape(n, b * d)
    x1 = _spmm(row, col, w, x0)
    x2 = 2.0 * _spmm(row, col, w, x1) - x0
    xs = jnp.stack([x0, x1, x2], axis=0)
    xs = xs.reshape(K, n, b, d).transpose(2, 1, 3, 0).reshape(b * n, d * K)
    out = _final_matmul(xs, weight, biases)
    return out.reshape(b, n, D)


# speedup vs baseline: 1.4752x; 1.3610x over previous
import functools

import jax
import jax.numpy as jnp
from jax import lax
from jax.experimental import pallas as pl
from jax.experimental.pallas import tpu as pltpu
from jax.experimental.pallas import tpu_sc as plsc

N = 49152
E = 393216
D = 128
B = 4
K = 3

NBLK = 32
BLKROWS = N // NBLK
NCORES = 2
NSUB = 16
CH = 64


def _spmm_body(x_hbm, ebp_hbm, ebw_hbm, bs_hbm, y_hbm,
               bs_v, pkbuf, wbuf, colb, rlb, rows, zbuf, acc):
    c = lax.axis_index("c")
    t = lax.axis_index("s")
    i16 = lax.iota(jnp.int32, 16)

    pltpu.sync_copy(bs_hbm, bs_v)

    def _zb(i, _):
        for g in range(8):
            zbuf[i, pl.ds(g * 16, 16)] = jnp.zeros((16,), jnp.float32)
        return 0
    lax.fori_loop(0, 64, _zb, 0)

    def _blk(bi, _):
        blk = 2 * bi + c
        base_row = blk * BLKROWS
        bsv = bs_v[pl.ds(blk, 16)]
        s0 = bsv[0]
        e0 = bsv[1]

        for z in range(6):
            pltpu.sync_copy(zbuf, acc.at[pl.ds(t * 384 + z * 64, 64)])
        plsc.subcore_barrier()

        cnt = e0 - s0
        share = ((cnt + 127) // 128) * 8
        a = s0 + t * share
        bt = jnp.minimum(e0, a + share)
        a8 = (a // 8) * 8
        nch = jnp.maximum(0, (bt - a8 + (CH - 1)) // CH)

        def _chunk(j, _):
            cb = a8 + j * CH
            pltpu.sync_copy(ebp_hbm.at[pl.ds(cb, CH)], pkbuf)
            pltpu.sync_copy(ebw_hbm.at[pl.ds(cb, CH)], wbuf.at[pl.ds(0, CH)])
            for g in range(4):
                ds16 = pl.ds(g * 16, 16)
                pk = pkbuf[ds16]
                gidx = i16 + (cb + g * 16)
                valid = (gidx >= a) & (gidx < bt)
                wbuf[ds16] = jnp.where(valid, wbuf[ds16], 0.0)
                colv = pk & 0xFFFF
                rlv = lax.shift_right_logical(pk, 16)
                for b in range(B):
                    colb[b, ds16] = colv + b * N
                    rlb[b, ds16] = rlv + b * BLKROWS
            for b in range(B):
                pltpu.sync_copy(x_hbm.at[colb.at[b]], rows.at[pl.ds(b * CH, CH)])

            def _scale(i, _):
                wv = wbuf[pl.ds(i, 16)][0]
                for b in range(B):
                    r = b * CH + i
                    for v in range(8):
                        dsv = pl.ds(v * 16, 16)
                        rows[r, dsv] = rows[r, dsv] * wv
                return 0
            lax.fori_loop(0, CH, _scale, 0)

            for b in range(B):
                pltpu.sync_copy(rows.at[pl.ds(b * CH, CH)],
                                acc.at[rlb.at[b]], add=True)
            return 0
        lax.fori_loop(0, nch, _chunk, 0)
        plsc.subcore_barrier()

        for b in range(B):
            pltpu.sync_copy(acc.at[pl.ds(b * BLKROWS + t * 96, 96)],
                            y_hbm.at[pl.ds(b * N + base_row + t * 96, 96)])
        plsc.subcore_barrier()
        return 0
    lax.fori_loop(0, NBLK // NCORES, _blk, 0)


@functools.partial(
    pl.kernel,
    out_type=jax.ShapeDtypeStruct((B * N, D), jnp.float32),
    mesh=plsc.VectorSubcoreMesh(core_axis_name="c", subcore_axis_name="s"),
    scratch_types=[
        pltpu.VMEM((48,), jnp.int32),
        pltpu.VMEM((CH,), jnp.int32),
        pltpu.VMEM((CH + 16,), jnp.float32),
        pltpu.VMEM((B, CH), jnp.int32),
        pltpu.VMEM((B, CH), jnp.int32),
        pltpu.VMEM((B * CH, D), jnp.float32),
        pltpu.VMEM((64, D), jnp.float32),
        pltpu.VMEM_SHARED((B * BLKROWS, D), jnp.float32),
    ],
)
def _spmm_sc(x_hbm, ebp_hbm, ebw_hbm, bs_hbm, y_hbm,
             bs_v, pkbuf, wbuf, colb, rlb, rows, zbuf, acc):
    _spmm_body(x_hbm, ebp_hbm, ebw_hbm, bs_hbm, y_hbm,
               bs_v, pkbuf, wbuf, colb, rlb, rows, zbuf, acc)


def _matmul_tc_body(x_ref, y1_ref, y2_ref, wa_ref, wb_ref, wc_ref, bias_ref,
                    o_ref):
    o_ref[...] = (
        jnp.dot(x_ref[...], wa_ref[...], preferred_element_type=jnp.float32)
        + jnp.dot(y1_ref[...], wb_ref[...], preferred_element_type=jnp.float32)
        + jnp.dot(y2_ref[...], wc_ref[...], preferred_element_type=jnp.float32)
        + bias_ref[...]
    )


def _final_matmul(xflat, y1, y2, wa, wb, wc, bias2d):
    m = xflat.shape[0]
    bm = 1024
    w_spec = pl.BlockSpec((D, D), lambda i: (0, 0))
    return pl.pallas_call(
        _matmul_tc_body,
        out_shape=jax.ShapeDtypeStruct((m, D), jnp.float32),
        grid=(m // bm,),
        in_specs=[
            pl.BlockSpec((bm, D), lambda i: (i, 0)),
            pl.BlockSpec((bm, D), lambda i: (i, 0)),
            pl.BlockSpec((bm, D), lambda i: (i, 0)),
            w_spec, w_spec, w_spec,
            pl.BlockSpec((1, D), lambda i: (0, 0)),
        ],
        out_specs=pl.BlockSpec((bm, D), lambda i: (i, 0)),
    )(xflat, y1, y2, wa, wb, wc, bias2d)


def kernel(x, edge_index, adj_values, weight, biases):
    row = edge_index[0]
    col = edge_index[1]
    v = adj_values

    deg = jnp.zeros((N,), v.dtype).at[row].add(v)
    w = v / jnp.clip(deg[row], 1e-6, None)

    packed = ((row % BLKROWS) << 16) | col
    perm = jnp.argsort(row)
    ebp = jnp.concatenate([packed[perm], jnp.zeros((CH,), jnp.int32)])
    ebw = jnp.concatenate([w[perm], jnp.zeros((CH,), jnp.float32)])
    rs = row[perm]
    bstart = jnp.searchsorted(
        rs, jnp.arange(NBLK + 1, dtype=jnp.int32) * BLKROWS).astype(jnp.int32)
    bstart = jnp.concatenate(
        [bstart, jnp.zeros((48 - (NBLK + 1),), jnp.int32)])

    xflat = x.reshape(B * N, D)
    y1 = _spmm_sc(xflat, ebp, ebw, bstart)
    y2 = _spmm_sc(y1, ebp, ebw, bstart)

    w0 = weight[0::K]
    w1 = weight[1::K]
    w2 = weight[2::K]
    out = _final_matmul(xflat, y1, y2, w0 - w2, w1, 2.0 * w2,
                        biases.reshape(1, D))
    return out.reshape(B, N, D)


# speedup vs baseline: 2.6283x; 1.7816x over previous
import functools

import jax
import jax.numpy as jnp
from jax import lax
from jax.experimental import pallas as pl
from jax.experimental.pallas import tpu as pltpu
from jax.experimental.pallas import tpu_sc as plsc

N = 49152
E = 393216
D = 128
B = 4
K = 3

NBLK = 32
BLKROWS = N // NBLK
NCORES = 2
NSUB = 16
CH = 64


def _spmm_body(x_hbm, ebp_hbm, ebw_hbm, rd_hbm, bs_hbm, y_hbm,
               bs_v, pkbuf, wbuf, rdbuf, rowg, colb, rlb, rows, zbuf, acc):
    c = lax.axis_index("c")
    t = lax.axis_index("s")
    i16 = lax.iota(jnp.int32, 16)

    pltpu.sync_copy(bs_hbm, bs_v)

    def _zb(i, _):
        for g in range(8):
            zbuf[i, pl.ds(g * 16, 16)] = jnp.zeros((16,), jnp.float32)
        return 0
    lax.fori_loop(0, 64, _zb, 0)

    def _blk(bi, _):
        blk = 2 * bi + c
        base_row = blk * BLKROWS
        bsv = bs_v[pl.ds(blk, 16)]
        s0 = bsv[0]
        e0 = bsv[1]

        for z in range(6):
            pltpu.sync_copy(zbuf, acc.at[pl.ds(t * 384 + z * 64, 64)])
        plsc.subcore_barrier()

        cnt = e0 - s0
        share = ((cnt + 127) // 128) * 8
        a = s0 + t * share
        bt = jnp.minimum(e0, a + share)
        a8 = (a // 8) * 8
        nch = jnp.maximum(0, (bt - a8 + (CH - 1)) // CH)

        def _chunk(j, _):
            cb = a8 + j * CH
            pltpu.sync_copy(ebp_hbm.at[pl.ds(cb, CH)], pkbuf)
            pltpu.sync_copy(ebw_hbm.at[pl.ds(cb, CH)], wbuf.at[pl.ds(0, CH)])
            for g in range(4):
                ds16 = pl.ds(g * 16, 16)
                pk = pkbuf[ds16]
                rlv = lax.shift_right_logical(pk, 16)
                rowg[ds16] = rlv + base_row
            pltpu.sync_copy(rd_hbm.at[rowg], rdbuf)
            for g in range(4):
                ds16 = pl.ds(g * 16, 16)
                pk = pkbuf[ds16]
                gidx = i16 + (cb + g * 16)
                valid = (gidx >= a) & (gidx < bt)
                wbuf[ds16] = jnp.where(valid, wbuf[ds16] * rdbuf[ds16], 0.0)
                colv = pk & 0xFFFF
                rlv = lax.shift_right_logical(pk, 16)
                for b in range(B):
                    colb[b, ds16] = colv + b * N
                    rlb[b, ds16] = rlv + b * BLKROWS
            for b in range(B):
                pltpu.sync_copy(x_hbm.at[colb.at[b]], rows.at[pl.ds(b * CH, CH)])

            def _scale(i, _):
                wv = wbuf[pl.ds(i, 16)][0]
                for b in range(B):
                    r = b * CH + i
                    for v in range(8):
                        dsv = pl.ds(v * 16, 16)
                        rows[r, dsv] = rows[r, dsv] * wv
                return 0
            lax.fori_loop(0, CH, _scale, 0)

            for b in range(B):
                pltpu.sync_copy(rows.at[pl.ds(b * CH, CH)],
                                acc.at[rlb.at[b]], add=True)
            return 0
        lax.fori_loop(0, nch, _chunk, 0)
        plsc.subcore_barrier()

        for b in range(B):
            pltpu.sync_copy(acc.at[pl.ds(b * BLKROWS + t * 96, 96)],
                            y_hbm.at[pl.ds(b * N + base_row + t * 96, 96)])
        plsc.subcore_barrier()
        return 0
    lax.fori_loop(0, NBLK // NCORES, _blk, 0)


@functools.partial(
    pl.kernel,
    out_type=jax.ShapeDtypeStruct((B * N, D), jnp.float32),
    mesh=plsc.VectorSubcoreMesh(core_axis_name="c", subcore_axis_name="s"),
    scratch_types=[
        pltpu.VMEM((48,), jnp.int32),
        pltpu.VMEM((CH,), jnp.int32),
        pltpu.VMEM((CH + 16,), jnp.float32),
        pltpu.VMEM((CH,), jnp.float32),
        pltpu.VMEM((CH,), jnp.int32),
        pltpu.VMEM((B, CH), jnp.int32),
        pltpu.VMEM((B, CH), jnp.int32),
        pltpu.VMEM((B * CH, D), jnp.float32),
        pltpu.VMEM((64, D), jnp.float32),
        pltpu.VMEM_SHARED((B * BLKROWS, D), jnp.float32),
    ],
)
def _spmm_sc(x_hbm, ebp_hbm, ebw_hbm, rd_hbm, bs_hbm, y_hbm,
             bs_v, pkbuf, wbuf, rdbuf, rowg, colb, rlb, rows, zbuf, acc):
    _spmm_body(x_hbm, ebp_hbm, ebw_hbm, rd_hbm, bs_hbm, y_hbm,
               bs_v, pkbuf, wbuf, rdbuf, rowg, colb, rlb, rows, zbuf, acc)


def _matmul_tc_body(x_ref, y1_ref, y2_ref, wa_ref, wb_ref, wc_ref, bias_ref,
                    o_ref):
    o_ref[...] = (
        jnp.dot(x_ref[...], wa_ref[...], preferred_element_type=jnp.float32)
        + jnp.dot(y1_ref[...], wb_ref[...], preferred_element_type=jnp.float32)
        + jnp.dot(y2_ref[...], wc_ref[...], preferred_element_type=jnp.float32)
        + bias_ref[...]
    )


def _final_matmul(xflat, y1, y2, wa, wb, wc, bias2d):
    m = xflat.shape[0]
    bm = 1024
    w_spec = pl.BlockSpec((D, D), lambda i: (0, 0))
    return pl.pallas_call(
        _matmul_tc_body,
        out_shape=jax.ShapeDtypeStruct((m, D), jnp.float32),
        grid=(m // bm,),
        in_specs=[
            pl.BlockSpec((bm, D), lambda i: (i, 0)),
            pl.BlockSpec((bm, D), lambda i: (i, 0)),
            pl.BlockSpec((bm, D), lambda i: (i, 0)),
            w_spec, w_spec, w_spec,
            pl.BlockSpec((1, D), lambda i: (0, 0)),
        ],
        out_specs=pl.BlockSpec((bm, D), lambda i: (i, 0)),
    )(xflat, y1, y2, wa, wb, wc, bias2d)


def kernel(x, edge_index, adj_values, weight, biases):
    row = edge_index[0]
    col = edge_index[1]
    v = adj_values

    deg = jnp.zeros((N,), v.dtype).at[row].add(v)
    rdeg = 1.0 / jnp.clip(deg, 1e-6, None)

    packed = ((row % BLKROWS) << 16) | col
    rs, ebp, ebw = jax.lax.sort((row, packed, v), num_keys=1)
    ebp = jnp.concatenate([ebp, jnp.zeros((CH,), jnp.int32)])
    ebw = jnp.concatenate([ebw, jnp.zeros((CH,), jnp.float32)])
    bstart = jnp.searchsorted(
        rs, jnp.arange(NBLK + 1, dtype=jnp.int32) * BLKROWS).astype(jnp.int32)
    bstart = jnp.concatenate(
        [bstart, jnp.zeros((48 - (NBLK + 1),), jnp.int32)])

    xflat = x.reshape(B * N, D)
    y1 = _spmm_sc(xflat, ebp, ebw, rdeg, bstart)
    y2 = _spmm_sc(y1, ebp, ebw, rdeg, bstart)

    w0 = weight[0::K]
    w1 = weight[1::K]
    w2 = weight[2::K]
    out = _final_matmul(xflat, y1, y2, w0 - w2, w1, 2.0 * w2,
                        biases.reshape(1, D))
    return out.reshape(B, N, D)


# speedup vs baseline: 3.9834x; 1.5156x over previous
import functools

import jax
import jax.numpy as jnp
from jax import lax
from jax.experimental import pallas as pl
from jax.experimental.pallas import tpu as pltpu
from jax.experimental.pallas import tpu_sc as plsc

N = 49152
E = 393216
D = 128
B = 4
K = 3

NBLK = 32
BLKROWS = N // NBLK
NCORES = 2
NSUB = 16
CH = 64


def _spmm_body(x_hbm, ebp_hbm, ebw_hbm, rd_hbm, bs_hbm, y_hbm,
               bs_v, pkbuf, wbuf, rdbuf, rowg, colb, rlb, rows, zbuf, acc,
               msem, rdsem, gsem, ssem):
    c = lax.axis_index("c")
    t = lax.axis_index("s")
    i16 = lax.iota(jnp.int32, 16)

    pltpu.sync_copy(bs_hbm, bs_v)

    def _zb(i, _):
        for g in range(8):
            zbuf[i, pl.ds(g * 16, 16)] = jnp.zeros((16,), jnp.float32)
        return 0
    lax.fori_loop(0, 64, _zb, 0)

    def _scatter_descs(p):
        return [
            pltpu.make_async_copy(rows.at[p].at[pl.ds(b * CH, CH)],
                                  acc.at[rlb.at[p].at[b]], ssem.at[p])
            for b in range(B)
        ]

    def _blk(bi, _):
        blk = 2 * bi + c
        base_row = blk * BLKROWS
        bsv = bs_v[pl.ds(blk, 16)]
        s0 = bsv[0]
        e0 = bsv[1]

        for z in range(6):
            pltpu.sync_copy(zbuf, acc.at[pl.ds(t * 384 + z * 64, 64)])
        plsc.subcore_barrier()

        cnt = e0 - s0
        share = ((cnt + 127) // 128) * 8
        a = s0 + t * share
        bt = jnp.minimum(e0, a + share)
        a8 = (a // 8) * 8
        nch = jnp.maximum(0, (bt - a8 + (CH - 1)) // CH)
        nchp = (nch + 1) // 2

        def _pair(jj, _):
            meta = {}
            for p in (0, 1):
                cb = a8 + (2 * jj + p) * CH

                @pl.when(jj >= 1)
                def _drain():
                    for d in _scatter_descs(p):
                        d.wait()

                meta[p] = [
                    pltpu.async_copy(ebp_hbm.at[pl.ds(cb, CH)],
                                     pkbuf.at[p], msem.at[p]),
                    pltpu.async_copy(ebw_hbm.at[pl.ds(cb, CH)],
                                     wbuf.at[p].at[pl.ds(0, CH)],
                                     msem.at[p]),
                ]
            rdd = {}
            gd = {}
            for p in (0, 1):
                cb = a8 + (2 * jj + p) * CH
                for d in meta[p]:
                    d.wait()
                for g in range(4):
                    ds16 = pl.ds(g * 16, 16)
                    pk = pkbuf[p, ds16]
                    colv = pk & 0xFFFF
                    rlv = lax.shift_right_logical(pk, 16)
                    rowg[p, ds16] = rlv + base_row
                    for b in range(B):
                        colb[p, b, ds16] = colv + b * N
                        rlb[p, b, ds16] = rlv + b * BLKROWS
                gd[p] = [
                    pltpu.async_copy(x_hbm.at[colb.at[p].at[b]],
                                     rows.at[p].at[pl.ds(b * CH, CH)],
                                     gsem.at[p])
                    for b in range(B)
                ]
                rdd[p] = pltpu.async_copy(rd_hbm.at[rowg.at[p]],
                                          rdbuf.at[p], rdsem.at[p])
            for p in (0, 1):
                cb = a8 + (2 * jj + p) * CH
                rdd[p].wait()
                for g in range(4):
                    ds16 = pl.ds(g * 16, 16)
                    gidx = i16 + (cb + g * 16)
                    valid = (gidx >= a) & (gidx < bt)
                    wbuf[p, ds16] = jnp.where(
                        valid, wbuf[p, ds16] * rdbuf[p, ds16], 0.0)
                for d in gd[p]:
                    d.wait()

                def _scale(i, _):
                    wv = wbuf[p, pl.ds(i, 16)][0]
                    for b in range(B):
                        r = b * CH + i
                        for v in range(8):
                            dsv = pl.ds(v * 16, 16)
                            rows[p, r, dsv] = rows[p, r, dsv] * wv
                    return 0
                lax.fori_loop(0, CH, _scale, 0)

                for d in _scatter_descs(p):
                    d.start(add=True)
            return 0
        lax.fori_loop(0, nchp, _pair, 0)

        @pl.when(nchp >= 1)
        def _final_drain():
            for p in (0, 1):
                for d in _scatter_descs(p):
                    d.wait()

        plsc.subcore_barrier()

        for b in range(B):
            pltpu.sync_copy(acc.at[pl.ds(b * BLKROWS + t * 96, 96)],
                            y_hbm.at[pl.ds(b * N + base_row + t * 96, 96)])
        plsc.subcore_barrier()
        return 0
    lax.fori_loop(0, NBLK // NCORES, _blk, 0)


@functools.partial(
    pl.kernel,
    out_type=jax.ShapeDtypeStruct((B * N, D), jnp.float32),
    mesh=plsc.VectorSubcoreMesh(core_axis_name="c", subcore_axis_name="s"),
    scratch_types=[
        pltpu.VMEM((48,), jnp.int32),
        pltpu.VMEM((2, CH), jnp.int32),
        pltpu.VMEM((2, CH + 16), jnp.float32),
        pltpu.VMEM((2, CH), jnp.float32),
        pltpu.VMEM((2, CH), jnp.int32),
        pltpu.VMEM((2, B, CH), jnp.int32),
        pltpu.VMEM((2, B, CH), jnp.int32),
        pltpu.VMEM((2, B * CH, D), jnp.float32),
        pltpu.VMEM((64, D), jnp.float32),
        pltpu.VMEM_SHARED((B * BLKROWS, D), jnp.float32),
        pltpu.SemaphoreType.DMA((2,)),
        pltpu.SemaphoreType.DMA((2,)),
        pltpu.SemaphoreType.DMA((2,)),
        pltpu.SemaphoreType.DMA((2,)),
    ],
)
def _spmm_sc(x_hbm, ebp_hbm, ebw_hbm, rd_hbm, bs_hbm, y_hbm,
             bs_v, pkbuf, wbuf, rdbuf, rowg, colb, rlb, rows, zbuf, acc,
             msem, rdsem, gsem, ssem):
    _spmm_body(x_hbm, ebp_hbm, ebw_hbm, rd_hbm, bs_hbm, y_hbm,
               bs_v, pkbuf, wbuf, rdbuf, rowg, colb, rlb, rows, zbuf, acc,
               msem, rdsem, gsem, ssem)


def _matmul_tc_body(x_ref, y1_ref, y2_ref, wa_ref, wb_ref, wc_ref, bias_ref,
                    o_ref):
    o_ref[...] = (
        jnp.dot(x_ref[...], wa_ref[...], preferred_element_type=jnp.float32)
        + jnp.dot(y1_ref[...], wb_ref[...], preferred_element_type=jnp.float32)
        + jnp.dot(y2_ref[...], wc_ref[...], preferred_element_type=jnp.float32)
        + bias_ref[...]
    )


def _final_matmul(xflat, y1, y2, wa, wb, wc, bias2d):
    m = xflat.shape[0]
    bm = 1024
    w_spec = pl.BlockSpec((D, D), lambda i: (0, 0))
    return pl.pallas_call(
        _matmul_tc_body,
        out_shape=jax.ShapeDtypeStruct((m, D), jnp.float32),
        grid=(m // bm,),
        in_specs=[
            pl.BlockSpec((bm, D), lambda i: (i, 0)),
            pl.BlockSpec((bm, D), lambda i: (i, 0)),
            pl.BlockSpec((bm, D), lambda i: (i, 0)),
            w_spec, w_spec, w_spec,
            pl.BlockSpec((1, D), lambda i: (0, 0)),
        ],
        out_specs=pl.BlockSpec((bm, D), lambda i: (i, 0)),
    )(xflat, y1, y2, wa, wb, wc, bias2d)


def kernel(x, edge_index, adj_values, weight, biases):
    row = edge_index[0]
    col = edge_index[1]
    v = adj_values

    deg = jnp.zeros((N,), v.dtype).at[row].add(v)
    rdeg = 1.0 / jnp.clip(deg, 1e-6, None)

    packed = ((row % BLKROWS) << 16) | col
    rs, ebp, ebw = jax.lax.sort((row, packed, v), num_keys=1)
    ebp = jnp.concatenate([ebp, jnp.zeros((2 * CH,), jnp.int32)])
    ebw = jnp.concatenate([ebw, jnp.zeros((2 * CH,), jnp.float32)])
    bstart = jnp.searchsorted(
        rs, jnp.arange(NBLK + 1, dtype=jnp.int32) * BLKROWS).astype(jnp.int32)
    bstart = jnp.concatenate(
        [bstart, jnp.zeros((48 - (NBLK + 1),), jnp.int32)])

    xflat = x.reshape(B * N, D)
    y1 = _spmm_sc(xflat, ebp, ebw, rdeg, bstart)
    y2 = _spmm_sc(y1, ebp, ebw, rdeg, bstart)

    w0 = weight[0::K]
    w1 = weight[1::K]
    w2 = weight[2::K]
    out = _final_matmul(xflat, y1, y2, w0 - w2, w1, 2.0 * w2,
                        biases.reshape(1, D))
    return out.reshape(B, N, D)


# speedup vs baseline: 4.0136x; 1.0076x over previous
import functools

import jax
import jax.numpy as jnp
from jax import lax
from jax.experimental import pallas as pl
from jax.experimental.pallas import tpu as pltpu
from jax.experimental.pallas import tpu_sc as plsc

N = 49152
E = 393216
D = 128
B = 4
K = 3

NBLK = 32
BLKROWS = N // NBLK
NCORES = 2
NSUB = 16
CH = 64
NG = CH // 16


def _spmm_body(x_hbm, ebp_hbm, ebw_hbm, rd_hbm, bs_hbm, y_hbm,
               bs_v, pkbuf, wbuf, rdbuf, rowg, colb, rlb, rows, zbuf, acc,
               msem, rdsem, gsem, ssem):
    c = lax.axis_index("c")
    t = lax.axis_index("s")
    i16 = lax.iota(jnp.int32, 16)

    pltpu.sync_copy(bs_hbm, bs_v)

    def _zb(i, _):
        for g in range(8):
            zbuf[i, pl.ds(g * 16, 16)] = jnp.zeros((16,), jnp.float32)
        return 0
    lax.fori_loop(0, 64, _zb, 0)

    def _scatter_descs(p):
        return [
            pltpu.make_async_copy(rows.at[p].at[pl.ds(b * CH, CH)],
                                  acc.at[rlb.at[p].at[b]], ssem.at[p])
            for b in range(B)
        ]

    def _blk(bi, _):
        blk = 2 * bi + c
        base_row = blk * BLKROWS
        bsv = bs_v[pl.ds(blk, 16)]
        s0 = bsv[0]
        e0 = bsv[1]

        for z in range(6):
            pltpu.sync_copy(zbuf, acc.at[pl.ds(t * 384 + z * 64, 64)])
        plsc.subcore_barrier()

        cnt = e0 - s0
        share = ((cnt + 127) // 128) * 8
        a = s0 + t * share
        bt = jnp.minimum(e0, a + share)
        a8 = (a // 8) * 8
        nch = jnp.maximum(0, (bt - a8 + (CH - 1)) // CH)
        nchp = (nch + 1) // 2

        def _pair(jj, _):
            meta = {}
            for p in (0, 1):
                cb = a8 + (2 * jj + p) * CH

                @pl.when(jj >= 1)
                def _drain():
                    for d in _scatter_descs(p):
                        d.wait()

                meta[p] = [
                    pltpu.async_copy(ebp_hbm.at[pl.ds(cb, CH)],
                                     pkbuf.at[p], msem.at[p]),
                    pltpu.async_copy(ebw_hbm.at[pl.ds(cb, CH)],
                                     wbuf.at[p].at[pl.ds(0, CH)],
                                     msem.at[p]),
                ]
            gd = {}
            rdd = {}
            for p in (0, 1):
                for d in meta[p]:
                    d.wait()
                for g in range(NG):
                    ds16 = pl.ds(g * 16, 16)
                    pk = pkbuf[p, ds16]
                    colv = pk & 0xFFFF
                    rlv = lax.shift_right_logical(pk, 16)
                    rowg[p, ds16] = rlv + base_row
                    for b in range(B):
                        colb[p, b, ds16] = colv + b * N
                        rlb[p, b, ds16] = rlv + b * BLKROWS
                gd[p] = [
                    pltpu.async_copy(x_hbm.at[colb.at[p].at[b]],
                                     rows.at[p].at[pl.ds(b * CH, CH)],
                                     gsem.at[p])
                    for b in range(B)
                ]
                rdd[p] = pltpu.async_copy(rd_hbm.at[rowg.at[p]],
                                          rdbuf.at[p], rdsem.at[p])
            for p in (0, 1):
                cb = a8 + (2 * jj + p) * CH
                rdd[p].wait()
                for g in range(NG):
                    ds16 = pl.ds(g * 16, 16)
                    gidx = i16 + (cb + g * 16)
                    valid = (gidx >= a) & (gidx < bt)
                    wbuf[p, ds16] = jnp.where(
                        valid, wbuf[p, ds16] * rdbuf[p, ds16], 0.0)
                for d in gd[p]:
                    d.wait()

                def _scale(i2, _):
                    for u in range(2):
                        i = 2 * i2 + u
                        wv = wbuf[p, pl.ds(i, 16)][0]
                        for b in range(B):
                            r = b * CH + i
                            for v in range(8):
                                dsv = pl.ds(v * 16, 16)
                                rows[p, r, dsv] = rows[p, r, dsv] * wv
                    return 0
                lax.fori_loop(0, CH // 2, _scale, 0)

                for d in _scatter_descs(p):
                    d.start(add=True)
            return 0
        lax.fori_loop(0, nchp, _pair, 0)

        @pl.when(nchp >= 1)
        def _final_drain():
            for p in (0, 1):
                for d in _scatter_descs(p):
                    d.wait()

        plsc.subcore_barrier()

        for b in range(B):
            pltpu.sync_copy(acc.at[pl.ds(b * BLKROWS + t * 96, 96)],
                            y_hbm.at[pl.ds(b * N + base_row + t * 96, 96)])
        plsc.subcore_barrier()
        return 0
    lax.fori_loop(0, NBLK // NCORES, _blk, 0)


@functools.partial(
    pl.kernel,
    out_type=jax.ShapeDtypeStruct((B * N, D), jnp.float32),
    mesh=plsc.VectorSubcoreMesh(core_axis_name="c", subcore_axis_name="s"),
    scratch_types=[
        pltpu.VMEM((48,), jnp.int32),
        pltpu.VMEM((2, CH), jnp.int32),
        pltpu.VMEM((2, CH + 16), jnp.float32),
        pltpu.VMEM((2, CH), jnp.float32),
        pltpu.VMEM((2, CH), jnp.int32),
        pltpu.VMEM((2, B, CH), jnp.int32),
        pltpu.VMEM((2, B, CH), jnp.int32),
        pltpu.VMEM((2, B * CH, D), jnp.float32),
        pltpu.VMEM((64, D), jnp.float32),
        pltpu.VMEM_SHARED((B * BLKROWS, D), jnp.float32),
        pltpu.SemaphoreType.DMA((2,)),
        pltpu.SemaphoreType.DMA((2,)),
        pltpu.SemaphoreType.DMA((2,)),
        pltpu.SemaphoreType.DMA((2,)),
    ],
)
def _spmm_sc(x_hbm, ebp_hbm, ebw_hbm, rd_hbm, bs_hbm, y_hbm,
             bs_v, pkbuf, wbuf, rdbuf, rowg, colb, rlb, rows, zbuf, acc,
             msem, rdsem, gsem, ssem):
    _spmm_body(x_hbm, ebp_hbm, ebw_hbm, rd_hbm, bs_hbm, y_hbm,
               bs_v, pkbuf, wbuf, rdbuf, rowg, colb, rlb, rows, zbuf, acc,
               msem, rdsem, gsem, ssem)


def _matmul_tc_body(x_ref, y1_ref, y2_ref, wa_ref, wb_ref, wc_ref, bias_ref,
                    o_ref):
    o_ref[...] = (
        jnp.dot(x_ref[...], wa_ref[...], preferred_element_type=jnp.float32)
        + jnp.dot(y1_ref[...], wb_ref[...], preferred_element_type=jnp.float32)
        + jnp.dot(y2_ref[...], wc_ref[...], preferred_element_type=jnp.float32)
        + bias_ref[...]
    )


def _final_matmul(xflat, y1, y2, wa, wb, wc, bias2d):
    m = xflat.shape[0]
    bm = 1024
    w_spec = pl.BlockSpec((D, D), lambda i: (0, 0))
    return pl.pallas_call(
        _matmul_tc_body,
        out_shape=jax.ShapeDtypeStruct((m, D), jnp.float32),
        grid=(m // bm,),
        in_specs=[
            pl.BlockSpec((bm, D), lambda i: (i, 0)),
            pl.BlockSpec((bm, D), lambda i: (i, 0)),
            pl.BlockSpec((bm, D), lambda i: (i, 0)),
            w_spec, w_spec, w_spec,
            pl.BlockSpec((1, D), lambda i: (0, 0)),
        ],
        out_specs=pl.BlockSpec((bm, D), lambda i: (i, 0)),
    )(xflat, y1, y2, wa, wb, wc, bias2d)


def kernel(x, edge_index, adj_values, weight, biases):
    row = edge_index[0]
    col = edge_index[1]
    v = adj_values

    deg = jnp.zeros((N,), v.dtype).at[row].add(v)
    rdeg = 1.0 / jnp.clip(deg, 1e-6, None)

    packed = ((row % BLKROWS) << 16) | col
    rs, ebp, ebw = jax.lax.sort((row, packed, v), num_keys=1)
    ebp = jnp.concatenate([ebp, jnp.zeros((2 * CH,), jnp.int32)])
    ebw = jnp.concatenate([ebw, jnp.zeros((2 * CH,), jnp.float32)])
    bstart = jnp.searchsorted(
        rs, jnp.arange(NBLK + 1, dtype=jnp.int32) * BLKROWS).astype(jnp.int32)
    bstart = jnp.concatenate(
        [bstart, jnp.zeros((48 - (NBLK + 1),), jnp.int32)])

    xflat = x.reshape(B * N, D)
    y1 = _spmm_sc(xflat, ebp, ebw, rdeg, bstart)
    y2 = _spmm_sc(y1, ebp, ebw, rdeg, bstart)

    w0 = weight[0::K]
    w1 = weight[1::K]
    w2 = weight[2::K]
    out = _final_matmul(xflat, y1, y2, w0 - w2, w1, 2.0 * w2,
                        biases.reshape(1, D))
    return out.reshape(B, N, D)
